# Initial kernel scaffold; baseline (speedup 1.0000x reference)
import jax
import jax.numpy as jnp
from jax.experimental import pallas as pl


def kernel(masks):
    raise NotImplementedError("write your pallas kernel here")


# speedup vs baseline: 4.3806x; 4.3806x over previous
import jax
import jax.numpy as jnp
from jax.experimental import pallas as pl
from jax.experimental.pallas import tpu as pltpu

_RSM_K = 35
_PFM_K = 31


def _su(x, d):
    return jnp.concatenate([x[d:, :], jnp.zeros((d, x.shape[1]), x.dtype)], axis=0)


def _sd(x, d):
    return jnp.concatenate([jnp.zeros((d, x.shape[1]), x.dtype), x[:-d, :]], axis=0)


def _col_sums(x):
    u1 = _su(x, 1)
    u2 = u1 + _su(u1, 1)
    u4 = u2 + _su(u2, 2)
    u8 = u4 + _su(u4, 4)
    u16 = u8 + _su(u8, 8)
    u17 = u16 + _su(x, 17)
    u15 = u16 - _su(x, 16)
    d1 = _sd(x, 1)
    d2 = d1 + _sd(d1, 1)
    d4 = d2 + _sd(d2, 2)
    d8 = d4 + _sd(d4, 4)
    d16 = d8 + _sd(d8, 8)
    d17 = d16 + _sd(x, 17)
    d15 = d16 - _sd(x, 16)
    c35 = d17 + x + u17
    c31 = d15 + x + u15
    return c35, c31


def _body(x_ref, a35_ref, a31_ref, rsm_ref, pfm_ref):
    x = x_ref[0]
    c35, c31 = _col_sums(x)
    r = jnp.dot(c35.astype(jnp.bfloat16), a35_ref[...],
                preferred_element_type=jnp.float32)
    rsm_ref[0] = r * (1.0 / (_RSM_K * _RSM_K))
    z = jnp.dot(c31.astype(jnp.bfloat16), a31_ref[...],
                preferred_element_type=jnp.float32)
    pfm_ref[0] = jnp.where(x > 0.5, 1, jnp.where(z > 0.5, 0, 2)).astype(jnp.int32)


def _band(n, r):
    i = jnp.arange(n)
    return (jnp.abs(i[:, None] - i[None, :]) <= r).astype(jnp.bfloat16)


def kernel(masks):
    b, _, h, w = masks.shape
    x = masks.reshape(b, h, w)
    a35 = _band(w, _RSM_K // 2)
    a31 = _band(w, _PFM_K // 2)
    rsm, pfm = pl.pallas_call(
        _body,
        grid=(b,),
        in_specs=[
            pl.BlockSpec((1, h, w), lambda i: (i, 0, 0)),
            pl.BlockSpec((w, w), lambda i: (0, 0)),
            pl.BlockSpec((w, w), lambda i: (0, 0)),
        ],
        out_specs=[
            pl.BlockSpec((1, h, w), lambda i: (i, 0, 0)),
            pl.BlockSpec((1, h, w), lambda i: (i, 0, 0)),
        ],
        out_shape=[
            jax.ShapeDtypeStruct((b, h, w), jnp.float32),
            jax.ShapeDtypeStruct((b, h, w), jnp.int32),
        ],
        compiler_params=pltpu.CompilerParams(
            dimension_semantics=("parallel",),
            vmem_limit_bytes=56 * 1024 * 1024,
        ),
        name="label_generator",
    )(x, a35, a31)
    return rsm.reshape(b, 1, h, w), pfm


# speedup vs baseline: 5.9086x; 1.3488x over previous
import jax
import jax.numpy as jnp
from jax.experimental import pallas as pl
from jax.experimental.pallas import tpu as pltpu

_RSM_K = 35
_PFM_K = 31


def _su(x, d):
    return jnp.concatenate([x[d:, :], jnp.zeros((d, x.shape[1]), x.dtype)], axis=0)


def _sd(x, d):
    return jnp.concatenate([jnp.zeros((d, x.shape[1]), x.dtype), x[:-d, :]], axis=0)


def _body(x_ref, a35_ref, a31_ref, rsm_ref, pfm_ref):
    x = x_ref[0]
    a35 = a35_ref[...]
    c35col = jnp.dot(a35, x.astype(jnp.bfloat16),
                     preferred_element_type=jnp.float32)
    u16 = _su(x, 16)
    d16 = _sd(x, 16)
    c31col = c35col - (u16 + _su(u16, 1) + d16 + _sd(d16, 1))
    r = jnp.dot(c35col.astype(jnp.bfloat16), a35,
                preferred_element_type=jnp.float32)
    rsm_ref[0] = r * (1.0 / (_RSM_K * _RSM_K))
    z = jnp.dot(c31col.astype(jnp.bfloat16), a31_ref[...],
                preferred_element_type=jnp.float32)
    pfm_ref[0] = jnp.where(x > 0.5, 1, jnp.where(z > 0.5, 0, 2)).astype(jnp.int32)


def _band(n, r):
    i = jnp.arange(n)
    return (jnp.abs(i[:, None] - i[None, :]) <= r).astype(jnp.bfloat16)


def kernel(masks):
    b, _, h, w = masks.shape
    x = masks.reshape(b, h, w)
    a35 = _band(w, _RSM_K // 2)
    a31 = _band(w, _PFM_K // 2)
    rsm, pfm = pl.pallas_call(
        _body,
        grid=(b,),
        in_specs=[
            pl.BlockSpec((1, h, w), lambda i: (i, 0, 0)),
            pl.BlockSpec((w, w), lambda i: (0, 0)),
            pl.BlockSpec((w, w), lambda i: (0, 0)),
        ],
        out_specs=[
            pl.BlockSpec((1, h, w), lambda i: (i, 0, 0)),
            pl.BlockSpec((1, h, w), lambda i: (i, 0, 0)),
        ],
        out_shape=[
            jax.ShapeDtypeStruct((b, h, w), jnp.float32),
            jax.ShapeDtypeStruct((b, h, w), jnp.int32),
        ],
        compiler_params=pltpu.CompilerParams(
            dimension_semantics=("parallel",),
            vmem_limit_bytes=56 * 1024 * 1024,
        ),
        name="label_generator",
    )(x, a35, a31)
    return rsm.reshape(b, 1, h, w), pfm
